# baseline (speedup 1.0000x reference)
import functools

import jax
import jax.numpy as jnp
from jax import lax
from jax.experimental import pallas as pl
from jax.experimental.pallas import tpu as pltpu
from jax.experimental.pallas import tpu_sc as plsc

B = 10000
NFEAT = 128
NHID = 128
NCLASS = 40
N0 = 10
N1 = 5

R = 400

NW = 32
OUT_F = B * N0 * NFEAT
FPW = OUT_F // NW
CH_ROWS = 125
CH_OUT = CH_ROWS * NFEAT
CH_IN = CH_OUT * N1
NCH = FPW // CH_OUT

_sc_mesh = plsc.VectorSubcoreMesh(core_axis_name="c", subcore_axis_name="s")


@functools.partial(
    pl.kernel,
    mesh=_sc_mesh,
    out_type=jax.ShapeDtypeStruct((OUT_F,), jnp.float32),
    scratch_types=[
        pltpu.VMEM((CH_IN,), jnp.float32),
        pltpu.VMEM((CH_OUT,), jnp.float32),
    ],
)
def _agg2_sc(x2_hbm, out_hbm, in_v, out_v):
    wid = lax.axis_index("s") * 2 + lax.axis_index("c")
    in_base = wid * (FPW * N1)
    out_base = wid * FPW

    def do_chunk(ci, carry):
        pltpu.sync_copy(x2_hbm.at[pl.ds(in_base + ci * CH_IN, CH_IN)], in_v)

        def do_row(rr, c2):
            ib = rr * (N1 * NFEAT)
            ob = rr * NFEAT
            for f in range(NFEAT // 16):
                o = 16 * f
                acc = (in_v[pl.ds(ib + o, 16)]
                       + in_v[pl.ds(ib + NFEAT + o, 16)]
                       + in_v[pl.ds(ib + 2 * NFEAT + o, 16)]
                       + in_v[pl.ds(ib + 3 * NFEAT + o, 16)]
                       + in_v[pl.ds(ib + 4 * NFEAT + o, 16)])
                out_v[pl.ds(ob + o, 16)] = acc * (1.0 / N1)
            return c2

        lax.fori_loop(0, CH_ROWS, do_row, 0)
        pltpu.sync_copy(out_v, out_hbm.at[pl.ds(out_base + ci * CH_OUT, CH_OUT)])
        return carry

    lax.fori_loop(0, NCH, do_chunk, 0)


def _gnn_block(x0_ref, x1_ref, agg2_ref, ws0_ref, wn0_ref, b0_ref,
               ws1_ref, wn1_ref, b1_ref, o_ref):
    x0b = x0_ref[...]
    x1b = x1_ref[...]
    agg2 = agg2_ref[...]

    ws0 = ws0_ref[...]
    wn0 = wn0_ref[...]
    b0 = b0_ref[...]

    h1 = jax.nn.relu(jnp.dot(x1b, ws0, preferred_element_type=jnp.float32)
                     + jnp.dot(agg2, wn0, preferred_element_type=jnp.float32)
                     + b0)

    rows = jax.lax.broadcasted_iota(jnp.int32, (R, N0 * R), 0)
    cols = jax.lax.broadcasted_iota(jnp.int32, (R, N0 * R), 1)
    P = jnp.where(cols // N0 == rows, 1.0 / N0, 0.0)

    agg1 = jnp.dot(P, x1b, preferred_element_type=jnp.float32)
    aggh = jnp.dot(P, h1, preferred_element_type=jnp.float32)

    h0 = jax.nn.relu(jnp.dot(x0b, ws0, preferred_element_type=jnp.float32)
                     + jnp.dot(agg1, wn0, preferred_element_type=jnp.float32)
                     + b0)

    out = (jnp.dot(h0, ws1_ref[...], preferred_element_type=jnp.float32)
           + jnp.dot(aggh, wn1_ref[...], preferred_element_type=jnp.float32)
           + b1_ref[...])

    m = jnp.max(out, axis=1, keepdims=True)
    s = out - m
    lse = jnp.log(jnp.sum(jnp.exp(s), axis=1, keepdims=True))
    o_ref[...] = s - lse


@jax.jit
def _run(x0, x1, x2f, W_self0, W_neigh0, b0, W_self1, W_neigh1, b1):
    agg2 = _agg2_sc(x2f).reshape(B * N0, NFEAT)
    grid = (B // R,)
    return pl.pallas_call(
        _gnn_block,
        grid=grid,
        in_specs=[
            pl.BlockSpec((R, NFEAT), lambda i: (i, 0)),
            pl.BlockSpec((N0 * R, NFEAT), lambda i: (i, 0)),
            pl.BlockSpec((N0 * R, NFEAT), lambda i: (i, 0)),
            pl.BlockSpec((NFEAT, NHID), lambda i: (0, 0)),
            pl.BlockSpec((NFEAT, NHID), lambda i: (0, 0)),
            pl.BlockSpec((1, NHID), lambda i: (0, 0)),
            pl.BlockSpec((NHID, NCLASS), lambda i: (0, 0)),
            pl.BlockSpec((NHID, NCLASS), lambda i: (0, 0)),
            pl.BlockSpec((1, NCLASS), lambda i: (0, 0)),
        ],
        out_specs=pl.BlockSpec((R, NCLASS), lambda i: (i, 0)),
        out_shape=jax.ShapeDtypeStruct((B, NCLASS), jnp.float32),
        compiler_params=pltpu.CompilerParams(
            dimension_semantics=("parallel",),
        ),
    )(x0, x1, agg2, W_self0, W_neigh0, b0, W_self1, W_neigh1, b1)


def kernel(x0, x1, x2, W_self0, W_neigh0, b0, W_self1, W_neigh1, b1):
    return _run(x0, x1, x2.reshape(-1), W_self0, W_neigh0, b0.reshape(1, NHID),
                W_self1, W_neigh1, b1.reshape(1, NCLASS))
